# Initial kernel scaffold; baseline (speedup 1.0000x reference)
import functools

import jax
import jax.numpy as jnp
from jax import lax
from jax.experimental import pallas as pl
from jax.experimental.pallas import tpu as pltpu
from jax.experimental.pallas import tpu_sc as plsc

B = 16384
NF = 26
ED = 64
NW = 32
BPW = B // NW
CHUNK_B = 4
CHUNK_L = CHUNK_B * NF
NCHUNK = BPW // CHUNK_B
LANES = 16
EC = ED // LANES

_MESH = plsc.VectorSubcoreMesh(core_axis_name="c", subcore_axis_name="s")


def _emb_bag_body(table_hbm, idx_hbm, w4e_hbm, out_hbm,
                  idx_v, rows_v, w4_v, out_v, sem):
    nc = lax.axis_size("c")
    wid = lax.axis_index("s") * nc + lax.axis_index("c")
    base_l = wid * BPW * NF

    pltpu.sync_copy(w4e_hbm, w4_v)

    def chunk_body(ch, carry):
        off = base_l + ch * CHUNK_L
        pltpu.sync_copy(idx_hbm.at[pl.ds(off, CHUNK_L)], idx_v)
        pltpu.async_copy(table_hbm.at[idx_v], rows_v, sem).wait()

        def fbody(f, accs):
            new = []
            for b in range(CHUNK_B):
                for c in range(EC):
                    r = rows_v[b * NF + f, pl.ds(c * LANES, LANES)]
                    w = w4_v[f, pl.ds(c * LANES, LANES)]
                    new.append(accs[b * EC + c] + r * w)
            return tuple(new)

        zero = jnp.zeros((LANES,), jnp.float32)
        accs = lax.fori_loop(
            0, NF, fbody, tuple(zero for _ in range(CHUNK_B * EC)))
        for b in range(CHUNK_B):
            tot = accs[b * EC]
            for c in range(1, EC):
                tot = tot + accs[b * EC + c]
            out_v[ch * CHUNK_B + b] = jnp.sum(tot)
        return carry

    lax.fori_loop(0, NCHUNK, chunk_body, 0)
    pltpu.sync_copy(out_v, out_hbm.at[pl.ds(wid * BPW, BPW)])


@functools.partial(
    pl.kernel,
    out_type=jax.ShapeDtypeStruct((B,), jnp.float32),
    mesh=_MESH,
    scratch_types=[
        pltpu.VMEM((CHUNK_L,), jnp.int32),
        pltpu.VMEM((CHUNK_L, ED), jnp.float32),
        pltpu.VMEM((NF, ED), jnp.float32),
        pltpu.VMEM((BPW,), jnp.float32),
        pltpu.SemaphoreType.DMA,
    ],
)
def _emb_bag(table_hbm, idx_hbm, w4e_hbm, out_hbm,
             idx_v, rows_v, w4_v, out_v, sem):
    _emb_bag_body(table_hbm, idx_hbm, w4e_hbm, out_hbm,
                  idx_v, rows_v, w4_v, out_v, sem)


MLP_BLK = 1024


def _mlp_body(x_ref, w1_ref, b1_ref, w2_ref, b2_ref, w3_ref, b3_ref,
              w4d_ref, b4_ref, out_ref):
    x = x_ref[...]
    h1 = jnp.maximum(
        jnp.dot(x, w1_ref[...], preferred_element_type=jnp.float32)
        + b1_ref[...], 0.0)
    h2 = jnp.maximum(
        jnp.dot(h1, w2_ref[...], preferred_element_type=jnp.float32)
        + b2_ref[...], 0.0)
    v = jnp.dot(w3_ref[...], w4d_ref[...],
                preferred_element_type=jnp.float32)
    c = (jnp.dot(b3_ref[...], w4d_ref[...],
                 preferred_element_type=jnp.float32)
         + b4_ref[...])
    out_ref[...] = (
        jnp.dot(h2, v, preferred_element_type=jnp.float32) + c)


def _mlp(dense_in, W1, b1, W2, b2, W3, b3, w4d, b4):
    full = lambda s: pl.BlockSpec(s, lambda i: (0, 0))
    return pl.pallas_call(
        _mlp_body,
        grid=(B // MLP_BLK,),
        in_specs=[
            pl.BlockSpec((MLP_BLK, 13), lambda i: (i, 0)),
            full((13, 256)), full((1, 256)),
            full((256, 256)), full((1, 256)),
            full((256, 256)), full((1, 256)),
            full((256, 1)), full((1, 1)),
        ],
        out_specs=pl.BlockSpec((MLP_BLK, 1), lambda i: (i, 0)),
        out_shape=jax.ShapeDtypeStruct((B, 1), jnp.float32),
    )(dense_in, W1, b1.reshape(1, 256), W2, b2.reshape(1, 256),
      W3, b3.reshape(1, 256), w4d, b4.reshape(1, 1))


def kernel(dense_in, sparse_idx, W1, b1, W2, b2, W3, b3, W4, b4, table):
    idx_flat = sparse_idx.astype(jnp.int32).reshape(-1)
    w4e = W4[: NF * ED, 0].reshape(NF, ED)
    w4d = W4[NF * ED :, :]
    emb = _emb_bag(table, idx_flat, w4e)
    dense = _mlp(dense_in, W1, b1, W2, b2, W3, b3, w4d, b4)
    return dense + emb[:, None]


# speedup vs baseline: 1.1148x; 1.1148x over previous
import functools

import jax
import jax.numpy as jnp
from jax import lax
from jax.experimental import pallas as pl
from jax.experimental.pallas import tpu as pltpu
from jax.experimental.pallas import tpu_sc as plsc

B = 16384
NF = 26
ED = 64
NW = 32
BPW = B // NW
CHUNK_B = 4
CHUNK_L = CHUNK_B * NF
NCHUNK = BPW // CHUNK_B
LANES = 16
EC = ED // LANES

GRP_B = 16
GRP_L = GRP_B * NF
NSUB = GRP_L // CHUNK_L
NGRP = BPW // GRP_B


def _emb_bag_body(table_hbm, idx_hbm, w4e_hbm, out_hbm,
                  idx_v, rows_v, w4_v, out_v, acc_v, sem):
    nc = lax.axis_size("c")
    wid = lax.axis_index("s") * nc + lax.axis_index("c")
    base_l = wid * BPW * NF

    pltpu.sync_copy(w4e_hbm, w4_v)

    def group_body(ch, carry):
        off = base_l + ch * GRP_L
        for g in range(NSUB):
            pltpu.sync_copy(
                idx_hbm.at[pl.ds(off + g * CHUNK_L, CHUNK_L)], idx_v.at[g])
        copies = [
            pltpu.async_copy(
                table_hbm.at[idx_v.at[g]],
                rows_v.at[pl.ds(g * CHUNK_L, CHUNK_L), :], sem)
            for g in range(NSUB)
        ]
        for cp in copies:
            cp.wait()

        def fbody(f, accs):
            ws = [w4_v[f, pl.ds(c * LANES, LANES)] for c in range(EC)]
            out = []
            for b in range(GRP_B):
                a = accs[b]
                for c in range(EC):
                    a = a + rows_v[b * NF + f, pl.ds(c * LANES, LANES)] * ws[c]
                out.append(a)
            return tuple(out)

        zero = jnp.zeros((LANES,), jnp.float32)
        accs = lax.fori_loop(0, NF, fbody, tuple(zero for _ in range(GRP_B)))
        for b in range(GRP_B):
            acc_v[b, :] = accs[b]
        lane = lax.iota(jnp.int32, LANES)
        vec = zero
        for j in range(LANES):
            col = plsc.load_gather(
                acc_v, [lane, jnp.full((LANES,), j, jnp.int32)])
            vec = vec + col
        out_v[pl.ds(ch * GRP_B, GRP_B)] = vec
        return carry

    lax.fori_loop(0, NGRP, group_body, 0)
    pltpu.sync_copy(out_v, out_hbm.at[pl.ds(wid * BPW, BPW)])


@functools.cache
def _emb_bag_fn():
    mesh = plsc.VectorSubcoreMesh(core_axis_name="c", subcore_axis_name="s")
    return pl.kernel(
        _emb_bag_body,
        out_type=jax.ShapeDtypeStruct((B,), jnp.float32),
        mesh=mesh,
        compiler_params=pltpu.CompilerParams(
            needs_layout_passes=False, use_tc_tiling_on_sc=False),
        scratch_types=[
            pltpu.VMEM((NSUB, CHUNK_L), jnp.int32),
            pltpu.VMEM((GRP_L, ED), jnp.float32),
            pltpu.VMEM((NF, ED), jnp.float32),
            pltpu.VMEM((BPW,), jnp.float32),
            pltpu.VMEM((LANES, LANES), jnp.float32),
            pltpu.SemaphoreType.DMA,
        ],
    )


MLP_BLK = 1024


def _mlp_body(x_ref, w1_ref, b1_ref, w2_ref, b2_ref, w3_ref, b3_ref,
              w4d_ref, b4_ref, out_ref):
    x = x_ref[...]
    h1 = jnp.maximum(
        jnp.dot(x, w1_ref[...], preferred_element_type=jnp.float32)
        + b1_ref[...], 0.0)
    h2 = jnp.maximum(
        jnp.dot(h1, w2_ref[...], preferred_element_type=jnp.float32)
        + b2_ref[...], 0.0)
    v = jnp.dot(w3_ref[...], w4d_ref[...],
                preferred_element_type=jnp.float32)
    c = (jnp.dot(b3_ref[...], w4d_ref[...],
                 preferred_element_type=jnp.float32)
         + b4_ref[...])
    out_ref[...] = (
        jnp.dot(h2, v, preferred_element_type=jnp.float32) + c)


def _mlp(dense_in, W1, b1, W2, b2, W3, b3, w4d, b4):
    full = lambda s: pl.BlockSpec(s, lambda i: (0, 0))
    return pl.pallas_call(
        _mlp_body,
        grid=(B // MLP_BLK,),
        in_specs=[
            pl.BlockSpec((MLP_BLK, 13), lambda i: (i, 0)),
            full((13, 256)), full((1, 256)),
            full((256, 256)), full((1, 256)),
            full((256, 256)), full((1, 256)),
            full((256, 1)), full((1, 1)),
        ],
        out_specs=pl.BlockSpec((MLP_BLK, 1), lambda i: (i, 0)),
        out_shape=jax.ShapeDtypeStruct((B, 1), jnp.float32),
    )(dense_in, W1, b1.reshape(1, 256), W2, b2.reshape(1, 256),
      W3, b3.reshape(1, 256), w4d, b4.reshape(1, 1))


def kernel(dense_in, sparse_idx, W1, b1, W2, b2, W3, b3, W4, b4, table):
    idx_flat = sparse_idx.astype(jnp.int32).reshape(-1)
    w4e = W4[: NF * ED, 0].reshape(NF, ED)
    w4d = W4[NF * ED :, :]
    emb = _emb_bag_fn()(table, idx_flat, w4e)
    dense = _mlp(dense_in, W1, b1, W2, b2, W3, b3, w4d, b4)
    return dense + emb[:, None]
